# baseline (speedup 1.0000x reference)
import functools

import jax
import jax.numpy as jnp
from jax import lax
from jax.experimental import pallas as pl
from jax.experimental.pallas import tpu as pltpu
from jax.experimental.pallas import tpu_sc as plsc

B = 16
IN_CH = 256
CH = 128
OUT_CH = 256
RES = 16
PIX = RES * RES
EMBED_DIM = 64
N_EMBED = 1024
N_PHYLO_CH = 64
FLAT_IN = N_PHYLO_CH * PIX
FLAT_CODE = 2048
NBI = 16
NBO = 16
BS_IN = FLAT_CODE // NBI
BS_OUT = FLAT_IN // NBO
QSTEP = NBI + 1
NROWS = B * 32


def _silu(v):
    return v * jax.nn.sigmoid(v)


def _front_kernel(x_ref, ciw_ref, cib_ref, lng_ref, lnb_ref,
                  wi_ref, bi_ref, cb_ref,
                  idx_ref, simg_ref,
                  flat_s, z_s):
    i = pl.program_id(0)

    @pl.when(i == 0)
    def _prologue():
        ciw = ciw_ref[...]
        cib = cib_ref[...]
        for b in range(B):
            sx = _silu(x_ref[b])
            h = jax.lax.dot_general(ciw, sx, (((1,), (0,)), ((), ())),
                                    preferred_element_type=jnp.float32) + cib
            hp = h[:N_PHYLO_CH]
            mu = jnp.mean(hp)
            var = jnp.mean((hp - mu) ** 2)
            hn = (hp - mu) * jax.lax.rsqrt(var + 1e-5)
            flat_s[b] = (hn * lng_ref[...] + lnb_ref[...]).astype(jnp.bfloat16)
            simg_ref[b] = _silu(h[N_PHYLO_CH:])

    @pl.when((i >= 1) & (i <= NBI))
    def _mlp_in():
        acc = jax.lax.dot_general(
            flat_s[...].reshape(B, FLAT_IN).astype(jnp.float32), wi_ref[...],
            (((1,), (1,)), ((), ())), preferred_element_type=jnp.float32)
        z_s[:, pl.ds((i - 1) * BS_IN, BS_IN)] = _silu(acc + bi_ref[0])

    @pl.when(i == QSTEP)
    def _argmin():
        cb = cb_ref[...]
        ones = jnp.ones((1, EMBED_DIM), jnp.float32)
        cb_sq = jax.lax.dot_general(ones, cb * cb, (((1,), (1,)), ((), ())),
                                    preferred_element_type=jnp.float32)
        iota = jax.lax.broadcasted_iota(jnp.int32, (128, N_EMBED), 1)
        for c in range(4):
            zc = z_s[4 * c:4 * c + 4]
            zf = jnp.transpose(zc.reshape(4, EMBED_DIM, 32),
                               (0, 2, 1)).reshape(128, EMBED_DIM)
            cross = jax.lax.dot_general(zf, cb, (((1,), (1,)), ((), ())),
                                        preferred_element_type=jnp.float32)
            d = cb_sq - 2.0 * cross
            dmin = jnp.min(d, axis=1, keepdims=True)
            idx = jnp.min(jnp.where(d <= dmin, iota, N_EMBED), axis=1,
                          keepdims=True)
            idx_ref[128 * c:128 * c + 128] = idx


def _make_sc_gather():
    info = plsc.get_sparse_core_info()
    nc, ns = info.num_cores, info.num_subcores
    nw = nc * ns
    b_per_w = NROWS // nw
    mesh = plsc.VectorSubcoreMesh(core_axis_name="c", subcore_axis_name="s")

    @functools.partial(
        pl.kernel, mesh=mesh,
        out_type=jax.ShapeDtypeStruct((NROWS, 128), jnp.float32),
        scratch_types=[
            pltpu.VMEM((b_per_w,), jnp.int32),
            pltpu.VMEM((b_per_w, 128), jnp.float32),
            pltpu.SemaphoreType.DMA,
        ],
    )
    def _sc_gather(table_hbm, idx_hbm, out_hbm, idx_v, rows_v, sem):
        wid = lax.axis_index("s") * nc + lax.axis_index("c")
        base = wid * b_per_w
        pltpu.sync_copy(idx_hbm.at[pl.ds(base, b_per_w)], idx_v)
        pltpu.async_copy(table_hbm.at[idx_v], rows_v, sem).wait()
        pltpu.sync_copy(rows_v, out_hbm.at[pl.ds(base, b_per_w)])

    return _sc_gather


def _mlp_out_kernel(zq_ref, w_ref, b_ref, h_ref):
    acc = jax.lax.dot_general(zq_ref[...], w_ref[...],
                              (((1,), (1,)), ((), ())),
                              preferred_element_type=jnp.float32)
    h_ref[...] = _silu(acc + b_ref[0])


def _epilogue_kernel(hout_ref, simg_ref, cow_ref, cob_ref, out_ref):
    w_p = cow_ref[:, :N_PHYLO_CH]
    w_i = cow_ref[:, N_PHYLO_CH:]
    cob = cob_ref[...]
    for b in range(B):
        sp = _silu(hout_ref[b])
        out_ref[b] = (
            jax.lax.dot_general(w_p, sp, (((1,), (0,)), ((), ())),
                                preferred_element_type=jnp.float32)
            + jax.lax.dot_general(w_i, simg_ref[b], (((1,), (0,)), ((), ())),
                                  preferred_element_type=jnp.float32)
            + cob)


def kernel(x, conv_in_w, conv_in_b, ln_g, ln_b, mlp_in_w, mlp_in_b,
           codebook, mlp_out_w, mlp_out_b, conv_out_w, conv_out_b):
    f32 = jnp.float32
    x_r = x.reshape(B, IN_CH, PIX)
    cib = conv_in_b.reshape(CH, 1)
    lng = ln_g.reshape(N_PHYLO_CH, PIX)
    lnb = ln_b.reshape(N_PHYLO_CH, PIX)
    b_in = mlp_in_b.reshape(NBI, 1, BS_IN)
    b_out = mlp_out_b.reshape(NBO, 1, BS_OUT)

    idx2, simg = pl.pallas_call(
        _front_kernel,
        grid=(QSTEP + 1,),
        in_specs=[
            pl.BlockSpec((B, IN_CH, PIX), lambda i: (0, 0, 0)),
            pl.BlockSpec((CH, IN_CH), lambda i: (0, 0)),
            pl.BlockSpec((CH, 1), lambda i: (0, 0)),
            pl.BlockSpec((N_PHYLO_CH, PIX), lambda i: (0, 0)),
            pl.BlockSpec((N_PHYLO_CH, PIX), lambda i: (0, 0)),
            pl.BlockSpec((BS_IN, FLAT_IN),
                         lambda i: (jnp.clip(i - 1, 0, NBI - 1), 0)),
            pl.BlockSpec((1, 1, BS_IN),
                         lambda i: (jnp.clip(i - 1, 0, NBI - 1), 0, 0)),
            pl.BlockSpec((N_EMBED, EMBED_DIM), lambda i: (0, 0)),
        ],
        out_specs=(pl.BlockSpec((NROWS, 1), lambda i: (0, 0)),
                   pl.BlockSpec((B, CH - N_PHYLO_CH, PIX),
                                lambda i: (0, 0, 0))),
        out_shape=(jax.ShapeDtypeStruct((NROWS, 1), jnp.int32),
                   jax.ShapeDtypeStruct((B, CH - N_PHYLO_CH, PIX), f32)),
        scratch_shapes=[
            pltpu.VMEM((B, N_PHYLO_CH, PIX), jnp.bfloat16),
            pltpu.VMEM((B, FLAT_CODE), f32),
        ],
        compiler_params=pltpu.CompilerParams(
            dimension_semantics=("arbitrary",)),
    )(x_r, conv_in_w, cib, lng, lnb, mlp_in_w, b_in, codebook)

    cb_pad = jnp.pad(codebook, ((0, 0), (0, 128 - EMBED_DIM)))
    zq_rows = _make_sc_gather()(cb_pad, idx2.reshape(NROWS))[:, :EMBED_DIM]
    zq = zq_rows.reshape(B, 32, EMBED_DIM).transpose(0, 2, 1).reshape(
        B, FLAT_CODE)

    hout = pl.pallas_call(
        _mlp_out_kernel,
        grid=(NBO,),
        in_specs=[
            pl.BlockSpec((B, FLAT_CODE), lambda i: (0, 0)),
            pl.BlockSpec((BS_OUT, FLAT_CODE), lambda i: (i, 0)),
            pl.BlockSpec((1, 1, BS_OUT), lambda i: (i, 0, 0)),
        ],
        out_specs=pl.BlockSpec((B, BS_OUT), lambda i: (0, i)),
        out_shape=jax.ShapeDtypeStruct((B, FLAT_IN), f32),
        compiler_params=pltpu.CompilerParams(
            dimension_semantics=("arbitrary",)),
    )(zq, mlp_out_w, b_out)

    cob = conv_out_b.reshape(OUT_CH, 1)
    out = pl.pallas_call(
        _epilogue_kernel,
        out_shape=jax.ShapeDtypeStruct((B, OUT_CH, PIX), f32),
    )(hout.reshape(B, N_PHYLO_CH, PIX), simg, conv_out_w, cob)
    return out.reshape(B, OUT_CH, RES, RES)


# speedup vs baseline: 1.2361x; 1.2361x over previous
import jax
import jax.numpy as jnp
from jax.experimental import pallas as pl
from jax.experimental.pallas import tpu as pltpu

B = 16
IN_CH = 256
CH = 128
OUT_CH = 256
RES = 16
PIX = RES * RES
EMBED_DIM = 64
N_EMBED = 1024
N_PHYLO_CH = 64
FLAT_IN = N_PHYLO_CH * PIX
FLAT_CODE = 2048
NBI = 16
NBO = 8
BS_IN = FLAT_CODE // NBI
BS_OUT = FLAT_IN // NBO
OSTEP = NBI + 1
ESTEP = OSTEP + NBO
STEPS = ESTEP + 8


def _silu(v):
    return v * jax.nn.sigmoid(v)


def _fused_kernel(x_ref, ciw_ref, cib_ref, lng_ref, lnb_ref,
                  wi_ref, bi_ref, cb_ref, wo_ref, bo_ref, cow_ref, cob_ref,
                  out_ref,
                  flat_s, z_s, hout_s, simg_s):
    i = pl.program_id(0)

    @pl.when(i == 0)
    def _prologue():
        ciw = ciw_ref[...]
        cib = cib_ref[...]
        for b in range(B):
            sx = _silu(x_ref[b])
            h = jax.lax.dot_general(ciw, sx, (((1,), (0,)), ((), ())),
                                    preferred_element_type=jnp.float32) + cib
            hp = h[:N_PHYLO_CH]
            mu = jnp.mean(hp)
            var = jnp.mean((hp - mu) ** 2)
            hn = (hp - mu) * jax.lax.rsqrt(var + 1e-5)
            flat_s[b] = (hn * lng_ref[...] + lnb_ref[...]).astype(jnp.bfloat16)
            simg_s[b] = _silu(h[N_PHYLO_CH:]).astype(jnp.bfloat16)

    @pl.when((i >= 1) & (i <= NBI))
    def _mlp_in():
        acc = jax.lax.dot_general(
            flat_s[...].reshape(B, FLAT_IN).astype(jnp.float32), wi_ref[...],
            (((1,), (1,)), ((), ())), preferred_element_type=jnp.float32)
        z_s[:, pl.ds((i - 1) * BS_IN, BS_IN)] = _silu(acc + bi_ref[0])

    @pl.when(i == OSTEP)
    def _quantize():
        cb = cb_ref[...]
        ones = jnp.ones((1, EMBED_DIM), jnp.float32)
        cb_sq = jax.lax.dot_general(ones, cb * cb, (((1,), (1,)), ((), ())),
                                    preferred_element_type=jnp.float32)
        iota = jax.lax.broadcasted_iota(jnp.int32, (128, N_EMBED), 1)
        for c in range(4):
            zc = z_s[4 * c:4 * c + 4]
            zf = jnp.transpose(zc.reshape(4, EMBED_DIM, 32),
                               (0, 2, 1)).reshape(128, EMBED_DIM)
            cross = jax.lax.dot_general(zf, cb, (((1,), (1,)), ((), ())),
                                        preferred_element_type=jnp.float32)
            d = cb_sq - 2.0 * cross
            dmin = jnp.min(d, axis=1, keepdims=True)
            idx = jnp.min(jnp.where(d <= dmin, iota, N_EMBED), axis=1,
                          keepdims=True)
            oh = (iota == idx).astype(jnp.float32)
            zq_rows = jax.lax.dot_general(oh, cb, (((1,), (0,)), ((), ())),
                                          preferred_element_type=jnp.float32)
            z_s[4 * c:4 * c + 4] = jnp.transpose(
                zq_rows.reshape(4, 32, EMBED_DIM),
                (0, 2, 1)).reshape(4, FLAT_CODE)

    @pl.when((i >= OSTEP) & (i < OSTEP + NBO))
    def _mlp_out():
        acc = jax.lax.dot_general(z_s[...], wo_ref[...],
                                  (((1,), (1,)), ((), ())),
                                  preferred_element_type=jnp.float32)
        h = _silu(acc + bo_ref[0])
        hout_s[i - OSTEP] = h.reshape(B, 8, PIX)

    @pl.when(i >= ESTEP)
    def _epilogue():
        w_p = cow_ref[:, :N_PHYLO_CH]
        w_i = cow_ref[:, N_PHYLO_CH:]
        cob = cob_ref[...]
        for bb in range(2):
            b = (i - ESTEP) * 2 + bb
            sp = _silu(hout_s[:, b].reshape(N_PHYLO_CH, PIX))
            out_ref[bb] = (
                jax.lax.dot_general(w_p, sp, (((1,), (0,)), ((), ())),
                                    preferred_element_type=jnp.float32)
                + jax.lax.dot_general(w_i, simg_s[b].astype(jnp.float32),
                                      (((1,), (0,)), ((), ())),
                                      preferred_element_type=jnp.float32)
                + cob)


def kernel(x, conv_in_w, conv_in_b, ln_g, ln_b, mlp_in_w, mlp_in_b,
           codebook, mlp_out_w, mlp_out_b, conv_out_w, conv_out_b):
    f32 = jnp.float32
    x_r = x.reshape(B, IN_CH, PIX)
    cib = conv_in_b.reshape(CH, 1)
    lng = ln_g.reshape(N_PHYLO_CH, PIX)
    lnb = ln_b.reshape(N_PHYLO_CH, PIX)
    b_in = mlp_in_b.reshape(NBI, 1, BS_IN)
    b_out = mlp_out_b.reshape(NBO, 1, BS_OUT)
    cob = conv_out_b.reshape(OUT_CH, 1)

    out = pl.pallas_call(
        _fused_kernel,
        grid=(STEPS,),
        in_specs=[
            pl.BlockSpec((B, IN_CH, PIX), lambda i: (0, 0, 0)),
            pl.BlockSpec((CH, IN_CH), lambda i: (0, 0)),
            pl.BlockSpec((CH, 1), lambda i: (0, 0)),
            pl.BlockSpec((N_PHYLO_CH, PIX), lambda i: (0, 0)),
            pl.BlockSpec((N_PHYLO_CH, PIX), lambda i: (0, 0)),
            pl.BlockSpec((BS_IN, FLAT_IN),
                         lambda i: (jnp.clip(i - 1, 0, NBI - 1), 0)),
            pl.BlockSpec((1, 1, BS_IN),
                         lambda i: (jnp.clip(i - 1, 0, NBI - 1), 0, 0)),
            pl.BlockSpec((N_EMBED, EMBED_DIM), lambda i: (0, 0)),
            pl.BlockSpec((BS_OUT, FLAT_CODE),
                         lambda i: (jnp.clip(i - OSTEP, 0, NBO - 1), 0)),
            pl.BlockSpec((1, 1, BS_OUT),
                         lambda i: (jnp.clip(i - OSTEP, 0, NBO - 1), 0, 0)),
            pl.BlockSpec((OUT_CH, CH), lambda i: (0, 0)),
            pl.BlockSpec((OUT_CH, 1), lambda i: (0, 0)),
        ],
        out_specs=pl.BlockSpec((2, OUT_CH, PIX),
                               lambda i: (jnp.clip(i - ESTEP, 0, 7), 0, 0)),
        out_shape=jax.ShapeDtypeStruct((B, OUT_CH, PIX), f32),
        scratch_shapes=[
            pltpu.VMEM((B, N_PHYLO_CH, PIX), jnp.bfloat16),
            pltpu.VMEM((B, FLAT_CODE), f32),
            pltpu.VMEM((NBO, B, 8, PIX), f32),
            pltpu.VMEM((B, CH - N_PHYLO_CH, PIX), jnp.bfloat16),
        ],
        compiler_params=pltpu.CompilerParams(
            dimension_semantics=("arbitrary",)),
    )(x_r, conv_in_w, cib, lng, lnb, mlp_in_w, b_in, codebook,
      mlp_out_w, b_out, conv_out_w, cob)
    return out.reshape(B, OUT_CH, RES, RES)
